# baseline (speedup 1.0000x reference)
import functools

import jax
import jax.numpy as jnp
from jax import lax
from jax.experimental import pallas as pl
from jax.experimental.pallas import tpu as pltpu
from jax.experimental.pallas import tpu_sc as plsc

T = 2048
H = 1024
E = 16
K = 2
F = 768
SF = 2816

RT = 256
ST = 2048
SFB = 256
B = 128
NT = 48
P = NT * B
NW = 32
TPW = T // NW


def _meta_body(h_ref, gw_ref, ids_ref, tw_ref, slots_ref, te_ref):
    logits = jnp.dot(h_ref[...], gw_ref[...], preferred_element_type=jnp.float32)
    iota_e = jax.lax.broadcasted_iota(jnp.int32, (T, E), 1)
    m1 = jnp.max(logits, axis=-1, keepdims=True)
    i1 = jnp.min(jnp.where(logits == m1, iota_e, E), axis=-1, keepdims=True)
    masked = jnp.where(iota_e == i1, -jnp.inf, logits)
    m2 = jnp.max(masked, axis=-1, keepdims=True)
    i2 = jnp.min(jnp.where(masked == m2, iota_e, E), axis=-1, keepdims=True)
    w1 = 1.0 / (1.0 + jnp.exp(m2 - m1))
    ids_ref[...] = jnp.concatenate([i1, i2], axis=1)
    tw_ref[...] = jnp.concatenate([w1, 1.0 - w1], axis=1)

    sel1 = iota_e == i1
    sel2 = iota_e == i2
    oh = (sel1 | sel2).astype(jnp.float32)

    r_io = jax.lax.broadcasted_iota(jnp.int32, (RT, RT), 0)
    c_io = jax.lax.broadcasted_iota(jnp.int32, (RT, RT), 1)
    tri = (r_io > c_io).astype(jnp.float32)

    ranks_rows = []
    base = jnp.zeros((1, E), jnp.float32)
    for b in range(T // RT):
        ohb = oh[b * RT:(b + 1) * RT]
        ranks_rows.append(jnp.dot(tri, ohb, preferred_element_type=jnp.float32) + base)
        base = base + jnp.sum(ohb, axis=0, keepdims=True)
    ranks = jnp.concatenate(ranks_rows, axis=0)
    counts = base

    ci = counts.astype(jnp.int32)
    pc = ((ci + B - 1) // B) * B
    pcf = pc.astype(jnp.float32)
    e_r = jax.lax.broadcasted_iota(jnp.int32, (E, E), 0)
    e_c = jax.lax.broadcasted_iota(jnp.int32, (E, E), 1)
    ut = (e_r <= e_c).astype(jnp.float32)
    cum_incl = jnp.dot(pcf, ut, preferred_element_type=jnp.float32)
    off = cum_incl - pcf

    slot1 = jnp.sum(jnp.where(sel1, off + ranks, 0.0), axis=-1, keepdims=True)
    slot2 = jnp.sum(jnp.where(sel2, off + ranks, 0.0), axis=-1, keepdims=True)
    slots_ref[...] = jnp.concatenate([slot1, slot2], axis=1).astype(jnp.int32)

    ti = jax.lax.broadcasted_iota(jnp.int32, (NT, E), 0)
    crossed = (ti * B) >= cum_incl.astype(jnp.int32)
    te = jnp.sum(crossed.astype(jnp.int32), axis=-1, keepdims=True)
    te_ref[...] = jnp.minimum(te, E - 1)


def _route_meta(hidden_states, gate_w):
    return pl.pallas_call(
        _meta_body,
        in_specs=[
            pl.BlockSpec((T, H), lambda: (0, 0)),
            pl.BlockSpec((H, E), lambda: (0, 0)),
        ],
        out_specs=[
            pl.BlockSpec((T, K), lambda: (0, 0)),
            pl.BlockSpec((T, K), lambda: (0, 0)),
            pl.BlockSpec((T, K), lambda: (0, 0)),
            pl.BlockSpec((NT, 1), lambda: (0, 0)),
        ],
        out_shape=[
            jax.ShapeDtypeStruct((T, K), jnp.int32),
            jax.ShapeDtypeStruct((T, K), jnp.float32),
            jax.ShapeDtypeStruct((T, K), jnp.int32),
            jax.ShapeDtypeStruct((NT, 1), jnp.int32),
        ],
    )(hidden_states, gate_w)


@functools.lru_cache(maxsize=None)
def _sc_dispatch_kernel():
    mesh = plsc.VectorSubcoreMesh(core_axis_name="c", subcore_axis_name="s")

    @functools.partial(
        pl.kernel,
        mesh=mesh,
        out_type=jax.ShapeDtypeStruct((P, H), jnp.float32),
        scratch_types=[
            pltpu.VMEM((K, TPW), jnp.int32),
            pltpu.VMEM((TPW, H), jnp.float32),
            pltpu.SemaphoreType.DMA,
            pltpu.SemaphoreType.DMA,
        ],
    )
    def dispatch(h_hbm, idx_hbm, xs_hbm, idx_v, xbuf, sem0, sem1):
        wid = lax.axis_index("s") * 2 + lax.axis_index("c")
        base = wid * TPW
        pltpu.sync_copy(h_hbm.at[pl.ds(base, TPW)], xbuf)
        pltpu.sync_copy(idx_hbm.at[wid], idx_v)
        c0 = pltpu.async_copy(xbuf, xs_hbm.at[idx_v.at[0]], sem0)
        c1 = pltpu.async_copy(xbuf, xs_hbm.at[idx_v.at[1]], sem1)
        c0.wait()
        c1.wait()

    return dispatch


def _sc_dispatch(hidden_states, idx3):
    return _sc_dispatch_kernel()(hidden_states, idx3)


@functools.lru_cache(maxsize=None)
def _sc_combine_kernel():
    mesh = plsc.VectorSubcoreMesh(core_axis_name="c", subcore_axis_name="s")

    @functools.partial(
        pl.kernel,
        mesh=mesh,
        out_type=[
            jax.ShapeDtypeStruct((T, H), jnp.float32),
            jax.ShapeDtypeStruct((T, H), jnp.float32),
        ],
        scratch_types=[
            pltpu.VMEM((K, TPW), jnp.int32),
            pltpu.VMEM((TPW, H), jnp.float32),
            pltpu.SemaphoreType.DMA,
        ],
    )
    def combine(y_hbm, idx_hbm, c0_hbm, c1_hbm, idx_v, ybuf, sem):
        wid = lax.axis_index("s") * 2 + lax.axis_index("c")
        base = wid * TPW
        pltpu.sync_copy(idx_hbm.at[wid], idx_v)
        pltpu.async_copy(y_hbm.at[idx_v.at[0]], ybuf, sem).wait()
        pltpu.sync_copy(ybuf, c0_hbm.at[pl.ds(base, TPW)])
        pltpu.async_copy(y_hbm.at[idx_v.at[1]], ybuf, sem).wait()
        pltpu.sync_copy(ybuf, c1_hbm.at[pl.ds(base, TPW)])

    return combine


def _sc_combine(ys, idx3):
    return _sc_combine_kernel()(ys, idx3)


def _group_body(te_ref, xs_ref, wg_ref, wu_ref, wd_ref, ys_ref):
    x = xs_ref[...]
    g = jnp.dot(x, wg_ref[0], preferred_element_type=jnp.float32)
    u = jnp.dot(x, wu_ref[0], preferred_element_type=jnp.float32)
    ys_ref[...] = jnp.dot(g * jax.lax.logistic(g) * u, wd_ref[0],
                          preferred_element_type=jnp.float32)


def _grouped_mlp(te_flat, xs, w_gate, w_up, w_down):
    grid_spec = pltpu.PrefetchScalarGridSpec(
        num_scalar_prefetch=1,
        grid=(NT,),
        in_specs=[
            pl.BlockSpec((B, H), lambda i, te: (i, 0)),
            pl.BlockSpec((1, H, F), lambda i, te: (te[i], 0, 0)),
            pl.BlockSpec((1, H, F), lambda i, te: (te[i], 0, 0)),
            pl.BlockSpec((1, F, H), lambda i, te: (te[i], 0, 0)),
        ],
        out_specs=pl.BlockSpec((B, H), lambda i, te: (i, 0)),
    )
    return pl.pallas_call(
        _group_body,
        grid_spec=grid_spec,
        out_shape=jax.ShapeDtypeStruct((P, H), jnp.float32),
        compiler_params=pltpu.CompilerParams(
            dimension_semantics=("arbitrary",)),
    )(te_flat, xs, w_gate, w_up, w_down)


def _shared_body(h_ref, wg_ref, wu_ref, wd_ref, sgw_ref, out_ref, gacc_ref):
    f = pl.program_id(1)
    x = h_ref[...]

    @pl.when(f == 0)
    def _():
        gacc_ref[...] = jnp.dot(x, sgw_ref[...], preferred_element_type=jnp.float32)

    g = jnp.dot(x, wg_ref[...], preferred_element_type=jnp.float32)
    u = jnp.dot(x, wu_ref[...], preferred_element_type=jnp.float32)
    part = jnp.dot(g * jax.lax.logistic(g) * u, wd_ref[...],
                   preferred_element_type=jnp.float32)
    acc = jnp.where(f == 0, 0.0, out_ref[...]) + part

    @pl.when(f == SF // SFB - 1)
    def _():
        out_ref[...] = acc * jax.lax.logistic(gacc_ref[...])

    @pl.when(f < SF // SFB - 1)
    def _():
        out_ref[...] = acc


def _shared(hidden_states, shared_w_gate, shared_w_up, shared_w_down, shared_gate_w):
    return pl.pallas_call(
        _shared_body,
        grid=(T // ST, SF // SFB),
        in_specs=[
            pl.BlockSpec((ST, H), lambda t, f: (t, 0)),
            pl.BlockSpec((H, SFB), lambda t, f: (0, f)),
            pl.BlockSpec((H, SFB), lambda t, f: (0, f)),
            pl.BlockSpec((SFB, H), lambda t, f: (f, 0)),
            pl.BlockSpec((H, 1), lambda t, f: (0, 0)),
        ],
        out_specs=pl.BlockSpec((ST, H), lambda t, f: (t, 0)),
        out_shape=jax.ShapeDtypeStruct((T, H), jnp.float32),
        scratch_shapes=[pltpu.VMEM((ST, 1), jnp.float32)],
        compiler_params=pltpu.CompilerParams(
            dimension_semantics=("parallel", "arbitrary")),
    )(hidden_states, shared_w_gate, shared_w_up, shared_w_down, shared_gate_w)


def _final_body(sh_ref, c0_ref, c1_ref, tw_ref, out_ref):
    tw = tw_ref[...]
    out_ref[...] = (sh_ref[...] + tw[:, 0:1] * c0_ref[...]
                    + tw[:, 1:2] * c1_ref[...])


def _final(shared_out, c0, c1, tw):
    return pl.pallas_call(
        _final_body,
        grid=(T // RT,),
        in_specs=[
            pl.BlockSpec((RT, H), lambda t: (t, 0)),
            pl.BlockSpec((RT, H), lambda t: (t, 0)),
            pl.BlockSpec((RT, H), lambda t: (t, 0)),
            pl.BlockSpec((RT, K), lambda t: (t, 0)),
        ],
        out_specs=pl.BlockSpec((RT, H), lambda t: (t, 0)),
        out_shape=jax.ShapeDtypeStruct((T, H), jnp.float32),
    )(shared_out, c0, c1, tw)


def kernel(hidden_states, gate_w, w_gate, w_up, w_down,
           shared_w_gate, shared_w_up, shared_w_down, shared_gate_w):
    topk_ids, topk_w, slots, te = _route_meta(hidden_states, gate_w)

    idx3 = slots.reshape(NW, TPW, K).transpose(0, 2, 1)
    te_flat = te.reshape(NT)

    xs = _sc_dispatch(hidden_states, idx3)
    ys = _grouped_mlp(te_flat, xs, w_gate, w_up, w_down)
    shared_out = _shared(hidden_states, shared_w_gate, shared_w_up,
                         shared_w_down, shared_gate_w)
    c0, c1 = _sc_combine(ys, idx3)
    out = _final(shared_out, c0, c1, topk_w)
    return out, topk_ids


# speedup vs baseline: 1.0380x; 1.0380x over previous
import functools

import jax
import jax.numpy as jnp
from jax import lax
from jax.experimental import pallas as pl
from jax.experimental.pallas import tpu as pltpu
from jax.experimental.pallas import tpu_sc as plsc

T = 2048
H = 1024
E = 16
K = 2
F = 768
SF = 2816

RT = 256
ST = 2048
SFB = 256
B = 256
NT = 32
P = NT * B
NW = 32
TPW = T // NW


def _meta_body(h_ref, gw_ref, ids_ref, tw_ref, slots_ref, te_ref):
    logits = jnp.dot(h_ref[...], gw_ref[...], preferred_element_type=jnp.float32)
    iota_e = jax.lax.broadcasted_iota(jnp.int32, (T, E), 1)
    m1 = jnp.max(logits, axis=-1, keepdims=True)
    i1 = jnp.min(jnp.where(logits == m1, iota_e, E), axis=-1, keepdims=True)
    masked = jnp.where(iota_e == i1, -jnp.inf, logits)
    m2 = jnp.max(masked, axis=-1, keepdims=True)
    i2 = jnp.min(jnp.where(masked == m2, iota_e, E), axis=-1, keepdims=True)
    w1 = 1.0 / (1.0 + jnp.exp(m2 - m1))
    ids_ref[...] = jnp.concatenate([i1, i2], axis=1)
    tw_ref[...] = jnp.concatenate([w1, 1.0 - w1], axis=1)

    sel1 = iota_e == i1
    sel2 = iota_e == i2
    oh = (sel1 | sel2).astype(jnp.float32)

    r_io = jax.lax.broadcasted_iota(jnp.int32, (RT, RT), 0)
    c_io = jax.lax.broadcasted_iota(jnp.int32, (RT, RT), 1)
    tri = (r_io > c_io).astype(jnp.float32)

    ranks_rows = []
    base = jnp.zeros((1, E), jnp.float32)
    for b in range(T // RT):
        ohb = oh[b * RT:(b + 1) * RT]
        ranks_rows.append(jnp.dot(tri, ohb, preferred_element_type=jnp.float32) + base)
        base = base + jnp.sum(ohb, axis=0, keepdims=True)
    ranks = jnp.concatenate(ranks_rows, axis=0)
    counts = base

    ci = counts.astype(jnp.int32)
    pc = ((ci + B - 1) // B) * B
    pcf = pc.astype(jnp.float32)
    e_r = jax.lax.broadcasted_iota(jnp.int32, (E, E), 0)
    e_c = jax.lax.broadcasted_iota(jnp.int32, (E, E), 1)
    ut = (e_r <= e_c).astype(jnp.float32)
    cum_incl = jnp.dot(pcf, ut, preferred_element_type=jnp.float32)
    off = cum_incl - pcf

    slot1 = jnp.sum(jnp.where(sel1, off + ranks, 0.0), axis=-1, keepdims=True)
    slot2 = jnp.sum(jnp.where(sel2, off + ranks, 0.0), axis=-1, keepdims=True)
    slots_ref[...] = jnp.concatenate([slot1, slot2], axis=1).astype(jnp.int32)

    ti = jax.lax.broadcasted_iota(jnp.int32, (NT, E), 0)
    crossed = (ti * B) >= cum_incl.astype(jnp.int32)
    te = jnp.sum(crossed.astype(jnp.int32), axis=-1, keepdims=True)
    te_ref[...] = jnp.minimum(te, E - 1)


def _route_meta(hidden_states, gate_w):
    return pl.pallas_call(
        _meta_body,
        in_specs=[
            pl.BlockSpec((T, H), lambda: (0, 0)),
            pl.BlockSpec((H, E), lambda: (0, 0)),
        ],
        out_specs=[
            pl.BlockSpec((T, K), lambda: (0, 0)),
            pl.BlockSpec((T, K), lambda: (0, 0)),
            pl.BlockSpec((T, K), lambda: (0, 0)),
            pl.BlockSpec((NT, 1), lambda: (0, 0)),
        ],
        out_shape=[
            jax.ShapeDtypeStruct((T, K), jnp.int32),
            jax.ShapeDtypeStruct((T, K), jnp.float32),
            jax.ShapeDtypeStruct((T, K), jnp.int32),
            jax.ShapeDtypeStruct((NT, 1), jnp.int32),
        ],
    )(hidden_states, gate_w)


@functools.lru_cache(maxsize=None)
def _sc_dispatch_kernel():
    mesh = plsc.VectorSubcoreMesh(core_axis_name="c", subcore_axis_name="s")

    @functools.partial(
        pl.kernel,
        mesh=mesh,
        out_type=jax.ShapeDtypeStruct((P, H), jnp.float32),
        scratch_types=[
            pltpu.VMEM((K, TPW), jnp.int32),
            pltpu.VMEM((TPW, H), jnp.float32),
            pltpu.SemaphoreType.DMA,
            pltpu.SemaphoreType.DMA,
        ],
    )
    def dispatch(h_hbm, idx_hbm, xs_hbm, idx_v, xbuf, sem0, sem1):
        wid = lax.axis_index("s") * 2 + lax.axis_index("c")
        base = wid * TPW
        pltpu.sync_copy(h_hbm.at[pl.ds(base, TPW)], xbuf)
        pltpu.sync_copy(idx_hbm.at[wid], idx_v)
        c0 = pltpu.async_copy(xbuf, xs_hbm.at[idx_v.at[0]], sem0)
        c1 = pltpu.async_copy(xbuf, xs_hbm.at[idx_v.at[1]], sem1)
        c0.wait()
        c1.wait()

    return dispatch


def _sc_dispatch(hidden_states, idx3):
    return _sc_dispatch_kernel()(hidden_states, idx3)


@functools.lru_cache(maxsize=None)
def _sc_combine_kernel():
    mesh = plsc.VectorSubcoreMesh(core_axis_name="c", subcore_axis_name="s")

    @functools.partial(
        pl.kernel,
        mesh=mesh,
        out_type=[
            jax.ShapeDtypeStruct((T, H), jnp.float32),
            jax.ShapeDtypeStruct((T, H), jnp.float32),
        ],
        scratch_types=[
            pltpu.VMEM((K, TPW), jnp.int32),
            pltpu.VMEM((TPW, H), jnp.float32),
            pltpu.SemaphoreType.DMA,
        ],
    )
    def combine(y_hbm, idx_hbm, c0_hbm, c1_hbm, idx_v, ybuf, sem):
        wid = lax.axis_index("s") * 2 + lax.axis_index("c")
        base = wid * TPW
        pltpu.sync_copy(idx_hbm.at[wid], idx_v)
        pltpu.async_copy(y_hbm.at[idx_v.at[0]], ybuf, sem).wait()
        pltpu.sync_copy(ybuf, c0_hbm.at[pl.ds(base, TPW)])
        pltpu.async_copy(y_hbm.at[idx_v.at[1]], ybuf, sem).wait()
        pltpu.sync_copy(ybuf, c1_hbm.at[pl.ds(base, TPW)])

    return combine


def _sc_combine(ys, idx3):
    return _sc_combine_kernel()(ys, idx3)


def _group_body(te_ref, xs_ref, wg_ref, wu_ref, wd_ref, ys_ref):
    x = xs_ref[...]
    g = jnp.dot(x, wg_ref[0], preferred_element_type=jnp.float32)
    u = jnp.dot(x, wu_ref[0], preferred_element_type=jnp.float32)
    ys_ref[...] = jnp.dot(g * jax.lax.logistic(g) * u, wd_ref[0],
                          preferred_element_type=jnp.float32)


def _grouped_mlp(te_flat, xs, w_gate, w_up, w_down):
    grid_spec = pltpu.PrefetchScalarGridSpec(
        num_scalar_prefetch=1,
        grid=(NT,),
        in_specs=[
            pl.BlockSpec((B, H), lambda i, te: (i, 0)),
            pl.BlockSpec((1, H, F), lambda i, te: (te[i], 0, 0)),
            pl.BlockSpec((1, H, F), lambda i, te: (te[i], 0, 0)),
            pl.BlockSpec((1, F, H), lambda i, te: (te[i], 0, 0)),
        ],
        out_specs=pl.BlockSpec((B, H), lambda i, te: (i, 0)),
    )
    return pl.pallas_call(
        _group_body,
        grid_spec=grid_spec,
        out_shape=jax.ShapeDtypeStruct((P, H), jnp.float32),
        compiler_params=pltpu.CompilerParams(
            dimension_semantics=("arbitrary",)),
    )(te_flat, xs, w_gate, w_up, w_down)


NFB = SF // SFB
NF1 = 6


def _shared_part_body(nf, f0, is_last, has_prev):
    def body(h_ref, wg_ref, wu_ref, wd_ref, sgw_ref, *rest):
        if has_prev:
            prev_ref, out_ref = rest
        else:
            (out_ref,) = rest
        f = pl.program_id(1)
        x = h_ref[...]
        g = jnp.dot(x, wg_ref[...], preferred_element_type=jnp.float32)
        u = jnp.dot(x, wu_ref[...], preferred_element_type=jnp.float32)
        part = jnp.dot(g * jax.lax.logistic(g) * u, wd_ref[...],
                       preferred_element_type=jnp.float32)
        init = prev_ref[...] if has_prev else 0.0
        acc = jnp.where(f == 0, init, out_ref[...]) + part

        if is_last:
            @pl.when(f == nf - 1)
            def _():
                gate = jnp.dot(x, sgw_ref[...], preferred_element_type=jnp.float32)
                out_ref[...] = acc * jax.lax.logistic(gate)

            @pl.when(f < nf - 1)
            def _():
                out_ref[...] = acc
        else:
            out_ref[...] = acc

    return body


def _shared_part(nf, f0, is_last, hidden_states, wg, wu, wd, sgw, prev=None):
    in_specs = [
        pl.BlockSpec((ST, H), lambda t, f: (t, 0)),
        pl.BlockSpec((H, SFB), lambda t, f: (0, f + f0)),
        pl.BlockSpec((H, SFB), lambda t, f: (0, f + f0)),
        pl.BlockSpec((SFB, H), lambda t, f: (f + f0, 0)),
        pl.BlockSpec((H, 1), lambda t, f: (0, 0)),
    ]
    args = [hidden_states, wg, wu, wd, sgw]
    if prev is not None:
        in_specs.append(pl.BlockSpec((ST, H), lambda t, f: (t, 0)))
        args.append(prev)
    return pl.pallas_call(
        _shared_part_body(nf, f0, is_last, prev is not None),
        grid=(T // ST, nf),
        in_specs=in_specs,
        out_specs=pl.BlockSpec((ST, H), lambda t, f: (t, 0)),
        out_shape=jax.ShapeDtypeStruct((T, H), jnp.float32),
        compiler_params=pltpu.CompilerParams(
            dimension_semantics=("parallel", "arbitrary")),
    )(*args)


def _final_body(sh_ref, c0_ref, c1_ref, tw_ref, out_ref):
    tw = tw_ref[...]
    out_ref[...] = (sh_ref[...] + tw[:, 0:1] * c0_ref[...]
                    + tw[:, 1:2] * c1_ref[...])


def _final(shared_out, c0, c1, tw):
    return pl.pallas_call(
        _final_body,
        grid=(T // RT,),
        in_specs=[
            pl.BlockSpec((RT, H), lambda t: (t, 0)),
            pl.BlockSpec((RT, H), lambda t: (t, 0)),
            pl.BlockSpec((RT, H), lambda t: (t, 0)),
            pl.BlockSpec((RT, K), lambda t: (t, 0)),
        ],
        out_specs=pl.BlockSpec((RT, H), lambda t: (t, 0)),
        out_shape=jax.ShapeDtypeStruct((T, H), jnp.float32),
    )(shared_out, c0, c1, tw)


def kernel(hidden_states, gate_w, w_gate, w_up, w_down,
           shared_w_gate, shared_w_up, shared_w_down, shared_gate_w):
    topk_ids, topk_w, slots, te = _route_meta(hidden_states, gate_w)

    idx3 = slots.reshape(NW, TPW, K).transpose(0, 2, 1)
    te_flat = te.reshape(NT)

    xs = _sc_dispatch(hidden_states, idx3)
    shared_p1 = _shared_part(NF1, 0, False, hidden_states, shared_w_gate,
                             shared_w_up, shared_w_down, shared_gate_w)
    ys = _grouped_mlp(te_flat, xs, w_gate, w_up, w_down)
    shared_out = _shared_part(NFB - NF1, NF1, True, hidden_states,
                              shared_w_gate, shared_w_up, shared_w_down,
                              shared_gate_w, shared_p1)
    c0, c1 = _sc_combine(ys, idx3)
    out = _final(shared_out, c0, c1, topk_w)
    return out, topk_ids


# speedup vs baseline: 1.1068x; 1.0663x over previous
import functools

import jax
import jax.numpy as jnp
from jax import lax
from jax.experimental import pallas as pl
from jax.experimental.pallas import tpu as pltpu
from jax.experimental.pallas import tpu_sc as plsc

T = 2048
H = 1024
E = 16
K = 2
F = 768
SF = 2816

RT = 256
ST = 2048
SFB = 256
B = 256
NT = 32
NTE = NT + 8
P = NT * B
NW = 32
TPW = T // NW


def _meta_body(h_ref, gw_ref, ids_ref, tw_ref, slots_ref, te_ref):
    logits = jnp.dot(h_ref[...], gw_ref[...], preferred_element_type=jnp.float32)
    iota_e = jax.lax.broadcasted_iota(jnp.int32, (T, E), 1)
    m1 = jnp.max(logits, axis=-1, keepdims=True)
    i1 = jnp.min(jnp.where(logits == m1, iota_e, E), axis=-1, keepdims=True)
    masked = jnp.where(iota_e == i1, -jnp.inf, logits)
    m2 = jnp.max(masked, axis=-1, keepdims=True)
    i2 = jnp.min(jnp.where(masked == m2, iota_e, E), axis=-1, keepdims=True)
    w1 = 1.0 / (1.0 + jnp.exp(m2 - m1))
    ids_ref[...] = jnp.concatenate([i1, i2], axis=1)
    tw_ref[...] = jnp.concatenate([w1, 1.0 - w1], axis=1)

    sel1 = iota_e == i1
    sel2 = iota_e == i2
    oh = (sel1 | sel2).astype(jnp.float32)

    r_io = jax.lax.broadcasted_iota(jnp.int32, (RT, RT), 0)
    c_io = jax.lax.broadcasted_iota(jnp.int32, (RT, RT), 1)
    tri = (r_io > c_io).astype(jnp.float32)

    ranks_rows = []
    base = jnp.zeros((1, E), jnp.float32)
    for b in range(T // RT):
        ohb = oh[b * RT:(b + 1) * RT]
        ranks_rows.append(jnp.dot(tri, ohb, preferred_element_type=jnp.float32) + base)
        base = base + jnp.sum(ohb, axis=0, keepdims=True)
    ranks = jnp.concatenate(ranks_rows, axis=0)
    counts = base

    ci = counts.astype(jnp.int32)
    pc = ((ci + B - 1) // B) * B
    pcf = pc.astype(jnp.float32)
    e_r = jax.lax.broadcasted_iota(jnp.int32, (E, E), 0)
    e_c = jax.lax.broadcasted_iota(jnp.int32, (E, E), 1)
    ut = (e_r <= e_c).astype(jnp.float32)
    cum_incl = jnp.dot(pcf, ut, preferred_element_type=jnp.float32)
    off = cum_incl - pcf

    slot1 = jnp.sum(jnp.where(sel1, off + ranks, 0.0), axis=-1, keepdims=True)
    slot2 = jnp.sum(jnp.where(sel2, off + ranks, 0.0), axis=-1, keepdims=True)
    slots_ref[...] = jnp.concatenate([slot1, slot2], axis=1).astype(jnp.int32)

    ti = jax.lax.broadcasted_iota(jnp.int32, (NTE, E), 0)
    crossed = (ti * B) >= cum_incl.astype(jnp.int32)
    te = jnp.minimum(jnp.sum(crossed.astype(jnp.int32), axis=-1, keepdims=True),
                     E - 1)
    n_used = cum_incl[:, E - 1:E].astype(jnp.int32) // B
    iota_r = jax.lax.broadcasted_iota(jnp.int32, (NTE, 1), 0)
    te_ref[...] = jnp.where(iota_r == NT, n_used, te)


def _route_meta(hidden_states, gate_w):
    return pl.pallas_call(
        _meta_body,
        in_specs=[
            pl.BlockSpec((T, H), lambda: (0, 0)),
            pl.BlockSpec((H, E), lambda: (0, 0)),
        ],
        out_specs=[
            pl.BlockSpec((T, K), lambda: (0, 0)),
            pl.BlockSpec((T, K), lambda: (0, 0)),
            pl.BlockSpec((T, K), lambda: (0, 0)),
            pl.BlockSpec((NTE, 1), lambda: (0, 0)),
        ],
        out_shape=[
            jax.ShapeDtypeStruct((T, K), jnp.int32),
            jax.ShapeDtypeStruct((T, K), jnp.float32),
            jax.ShapeDtypeStruct((T, K), jnp.int32),
            jax.ShapeDtypeStruct((NTE, 1), jnp.int32),
        ],
    )(hidden_states, gate_w)


@functools.lru_cache(maxsize=None)
def _sc_dispatch_kernel():
    mesh = plsc.VectorSubcoreMesh(core_axis_name="c", subcore_axis_name="s")

    @functools.partial(
        pl.kernel,
        mesh=mesh,
        out_type=jax.ShapeDtypeStruct((P, H), jnp.float32),
        scratch_types=[
            pltpu.VMEM((K, TPW), jnp.int32),
            pltpu.VMEM((TPW, H), jnp.float32),
            pltpu.SemaphoreType.DMA,
            pltpu.SemaphoreType.DMA,
        ],
    )
    def dispatch(h_hbm, idx_hbm, xs_hbm, idx_v, xbuf, sem0, sem1):
        wid = lax.axis_index("s") * 2 + lax.axis_index("c")
        base = wid * TPW
        pltpu.sync_copy(h_hbm.at[pl.ds(base, TPW)], xbuf)
        pltpu.sync_copy(idx_hbm.at[wid], idx_v)
        c0 = pltpu.async_copy(xbuf, xs_hbm.at[idx_v.at[0]], sem0)
        c1 = pltpu.async_copy(xbuf, xs_hbm.at[idx_v.at[1]], sem1)
        c0.wait()
        c1.wait()

    return dispatch


def _sc_dispatch(hidden_states, idx3):
    return _sc_dispatch_kernel()(hidden_states, idx3)


@functools.lru_cache(maxsize=None)
def _sc_combine_kernel():
    mesh = plsc.VectorSubcoreMesh(core_axis_name="c", subcore_axis_name="s")

    @functools.partial(
        pl.kernel,
        mesh=mesh,
        out_type=[
            jax.ShapeDtypeStruct((T, H), jnp.float32),
            jax.ShapeDtypeStruct((T, H), jnp.float32),
        ],
        scratch_types=[
            pltpu.VMEM((K, TPW), jnp.int32),
            pltpu.VMEM((TPW, H), jnp.float32),
            pltpu.SemaphoreType.DMA,
        ],
    )
    def combine(y_hbm, idx_hbm, c0_hbm, c1_hbm, idx_v, ybuf, sem):
        wid = lax.axis_index("s") * 2 + lax.axis_index("c")
        base = wid * TPW
        pltpu.sync_copy(idx_hbm.at[wid], idx_v)
        pltpu.async_copy(y_hbm.at[idx_v.at[0]], ybuf, sem).wait()
        pltpu.sync_copy(ybuf, c0_hbm.at[pl.ds(base, TPW)])
        pltpu.async_copy(y_hbm.at[idx_v.at[1]], ybuf, sem).wait()
        pltpu.sync_copy(ybuf, c1_hbm.at[pl.ds(base, TPW)])

    return combine


def _sc_combine(ys, idx3):
    return _sc_combine_kernel()(ys, idx3)


def _group_body(te_ref, xs_ref, wg_ref, wu_ref, wd_ref, ys_ref):
    @pl.when(pl.program_id(0) < te_ref[NT])
    def _():
        x = xs_ref[...]
        g = jnp.dot(x, wg_ref[0], preferred_element_type=jnp.float32)
        u = jnp.dot(x, wu_ref[0], preferred_element_type=jnp.float32)
        ys_ref[...] = jnp.dot(g * jax.lax.logistic(g) * u, wd_ref[0],
                              preferred_element_type=jnp.float32)


def _grouped_mlp(te_flat, xs, w_gate, w_up, w_down):
    grid_spec = pltpu.PrefetchScalarGridSpec(
        num_scalar_prefetch=1,
        grid=(NT,),
        in_specs=[
            pl.BlockSpec((B, H), lambda i, te: (i, 0)),
            pl.BlockSpec((1, H, F), lambda i, te: (te[i], 0, 0)),
            pl.BlockSpec((1, H, F), lambda i, te: (te[i], 0, 0)),
            pl.BlockSpec((1, F, H), lambda i, te: (te[i], 0, 0)),
        ],
        out_specs=pl.BlockSpec((B, H), lambda i, te: (i, 0)),
    )
    return pl.pallas_call(
        _group_body,
        grid_spec=grid_spec,
        out_shape=jax.ShapeDtypeStruct((P, H), jnp.float32),
        compiler_params=pltpu.CompilerParams(
            dimension_semantics=("arbitrary",)),
    )(te_flat, xs, w_gate, w_up, w_down)


NFB = SF // SFB
NF1 = 6


def _shared_part_body(nf, f0, is_last, has_prev):
    def body(h_ref, wg_ref, wu_ref, wd_ref, sgw_ref, *rest):
        if has_prev:
            prev_ref, out_ref = rest
        else:
            (out_ref,) = rest
        f = pl.program_id(1)
        x = h_ref[...]
        g = jnp.dot(x, wg_ref[...], preferred_element_type=jnp.float32)
        u = jnp.dot(x, wu_ref[...], preferred_element_type=jnp.float32)
        part = jnp.dot(g * jax.lax.logistic(g) * u, wd_ref[...],
                       preferred_element_type=jnp.float32)
        init = prev_ref[...] if has_prev else 0.0
        acc = jnp.where(f == 0, init, out_ref[...]) + part

        if is_last:
            @pl.when(f == nf - 1)
            def _():
                gate = jnp.dot(x, sgw_ref[...], preferred_element_type=jnp.float32)
                out_ref[...] = acc * jax.lax.logistic(gate)

            @pl.when(f < nf - 1)
            def _():
                out_ref[...] = acc
        else:
            out_ref[...] = acc

    return body


def _shared_part(nf, f0, is_last, hidden_states, wg, wu, wd, sgw, prev=None):
    in_specs = [
        pl.BlockSpec((ST, H), lambda t, f: (t, 0)),
        pl.BlockSpec((H, SFB), lambda t, f: (0, f + f0)),
        pl.BlockSpec((H, SFB), lambda t, f: (0, f + f0)),
        pl.BlockSpec((SFB, H), lambda t, f: (f + f0, 0)),
        pl.BlockSpec((H, 1), lambda t, f: (0, 0)),
    ]
    args = [hidden_states, wg, wu, wd, sgw]
    if prev is not None:
        in_specs.append(pl.BlockSpec((ST, H), lambda t, f: (t, 0)))
        args.append(prev)
    return pl.pallas_call(
        _shared_part_body(nf, f0, is_last, prev is not None),
        grid=(T // ST, nf),
        in_specs=in_specs,
        out_specs=pl.BlockSpec((ST, H), lambda t, f: (t, 0)),
        out_shape=jax.ShapeDtypeStruct((T, H), jnp.float32),
        compiler_params=pltpu.CompilerParams(
            dimension_semantics=("parallel", "arbitrary")),
    )(*args)


def _final_body(sh_ref, c0_ref, c1_ref, tw_ref, out_ref):
    tw = tw_ref[...]
    out_ref[...] = (sh_ref[...] + tw[:, 0:1] * c0_ref[...]
                    + tw[:, 1:2] * c1_ref[...])


def _final(shared_out, c0, c1, tw):
    return pl.pallas_call(
        _final_body,
        grid=(T // RT,),
        in_specs=[
            pl.BlockSpec((RT, H), lambda t: (t, 0)),
            pl.BlockSpec((RT, H), lambda t: (t, 0)),
            pl.BlockSpec((RT, H), lambda t: (t, 0)),
            pl.BlockSpec((RT, K), lambda t: (t, 0)),
        ],
        out_specs=pl.BlockSpec((RT, H), lambda t: (t, 0)),
        out_shape=jax.ShapeDtypeStruct((T, H), jnp.float32),
    )(shared_out, c0, c1, tw)


def kernel(hidden_states, gate_w, w_gate, w_up, w_down,
           shared_w_gate, shared_w_up, shared_w_down, shared_gate_w):
    topk_ids, topk_w, slots, te = _route_meta(hidden_states, gate_w)

    idx3 = slots.reshape(NW, TPW, K).transpose(0, 2, 1)
    te_flat = te.reshape(NTE)

    xs = _sc_dispatch(hidden_states, idx3)
    ys = _grouped_mlp(te_flat, xs, w_gate, w_up, w_down)
    shared_out = _shared_part(NFB, 0, True, hidden_states, shared_w_gate,
                              shared_w_up, shared_w_down, shared_gate_w)
    c0, c1 = _sc_combine(ys, idx3)
    out = _final(shared_out, c0, c1, topk_w)
    return out, topk_ids


# speedup vs baseline: 1.1115x; 1.0042x over previous
import functools

import jax
import jax.numpy as jnp
from jax import lax
from jax.experimental import pallas as pl
from jax.experimental.pallas import tpu as pltpu
from jax.experimental.pallas import tpu_sc as plsc

T = 2048
H = 1024
E = 16
K = 2
F = 768
SF = 2816

RT = 256
ST = 2048
SFB = 256
B = 256
NT = 32
NTE = NT + 8
P = NT * B
NW = 32
TPW = T // NW


def _meta_body(h_ref, gw_ref, ids_ref, tw_ref, slots_ref, te_ref):
    logits = jnp.dot(h_ref[...], gw_ref[...], preferred_element_type=jnp.float32)
    iota_e = jax.lax.broadcasted_iota(jnp.int32, (T, E), 1)
    m1 = jnp.max(logits, axis=-1, keepdims=True)
    i1 = jnp.min(jnp.where(logits == m1, iota_e, E), axis=-1, keepdims=True)
    masked = jnp.where(iota_e == i1, -jnp.inf, logits)
    m2 = jnp.max(masked, axis=-1, keepdims=True)
    i2 = jnp.min(jnp.where(masked == m2, iota_e, E), axis=-1, keepdims=True)
    w1 = 1.0 / (1.0 + jnp.exp(m2 - m1))
    ids_ref[...] = jnp.concatenate([i1, i2], axis=1)
    tw_ref[...] = jnp.concatenate([w1, 1.0 - w1], axis=1)

    sel1 = iota_e == i1
    sel2 = iota_e == i2
    oh = (sel1 | sel2).astype(jnp.float32)

    r_io = jax.lax.broadcasted_iota(jnp.int32, (RT, RT), 0)
    c_io = jax.lax.broadcasted_iota(jnp.int32, (RT, RT), 1)
    tri = (r_io > c_io).astype(jnp.float32)

    ranks_rows = []
    base = jnp.zeros((1, E), jnp.float32)
    for b in range(T // RT):
        ohb = oh[b * RT:(b + 1) * RT]
        ranks_rows.append(jnp.dot(tri, ohb, preferred_element_type=jnp.float32) + base)
        base = base + jnp.sum(ohb, axis=0, keepdims=True)
    ranks = jnp.concatenate(ranks_rows, axis=0)
    counts = base

    ci = counts.astype(jnp.int32)
    pc = ((ci + B - 1) // B) * B
    pcf = pc.astype(jnp.float32)
    e_r = jax.lax.broadcasted_iota(jnp.int32, (E, E), 0)
    e_c = jax.lax.broadcasted_iota(jnp.int32, (E, E), 1)
    ut = (e_r <= e_c).astype(jnp.float32)
    cum_incl = jnp.dot(pcf, ut, preferred_element_type=jnp.float32)
    off = cum_incl - pcf

    slot1 = jnp.sum(jnp.where(sel1, off + ranks, 0.0), axis=-1, keepdims=True)
    slot2 = jnp.sum(jnp.where(sel2, off + ranks, 0.0), axis=-1, keepdims=True)
    slots_ref[...] = jnp.concatenate([slot1, slot2], axis=1).astype(jnp.int32)

    ti = jax.lax.broadcasted_iota(jnp.int32, (NTE, E), 0)
    crossed = (ti * B) >= cum_incl.astype(jnp.int32)
    te = jnp.minimum(jnp.sum(crossed.astype(jnp.int32), axis=-1, keepdims=True),
                     E - 1)
    n_used = cum_incl[:, E - 1:E].astype(jnp.int32) // B
    iota_r = jax.lax.broadcasted_iota(jnp.int32, (NTE, 1), 0)
    te_ref[...] = jnp.where(iota_r == NT, n_used, te)


def _route_meta(hidden_states, gate_w):
    return pl.pallas_call(
        _meta_body,
        in_specs=[
            pl.BlockSpec((T, H), lambda: (0, 0)),
            pl.BlockSpec((H, E), lambda: (0, 0)),
        ],
        out_specs=[
            pl.BlockSpec((T, K), lambda: (0, 0)),
            pl.BlockSpec((T, K), lambda: (0, 0)),
            pl.BlockSpec((T, K), lambda: (0, 0)),
            pl.BlockSpec((NTE, 1), lambda: (0, 0)),
        ],
        out_shape=[
            jax.ShapeDtypeStruct((T, K), jnp.int32),
            jax.ShapeDtypeStruct((T, K), jnp.float32),
            jax.ShapeDtypeStruct((T, K), jnp.int32),
            jax.ShapeDtypeStruct((NTE, 1), jnp.int32),
        ],
    )(hidden_states, gate_w)


@functools.lru_cache(maxsize=None)
def _sc_dispatch_kernel():
    mesh = plsc.VectorSubcoreMesh(core_axis_name="c", subcore_axis_name="s")

    @functools.partial(
        pl.kernel,
        mesh=mesh,
        out_type=jax.ShapeDtypeStruct((P, H), jnp.float32),
        scratch_types=[
            pltpu.VMEM((K, TPW), jnp.int32),
            pltpu.VMEM((TPW, H), jnp.float32),
            pltpu.SemaphoreType.DMA,
            pltpu.SemaphoreType.DMA,
        ],
    )
    def dispatch(h_hbm, idx_hbm, xs_hbm, idx_v, xbuf, sem0, sem1):
        wid = lax.axis_index("s") * 2 + lax.axis_index("c")
        base = wid * TPW
        pltpu.sync_copy(h_hbm.at[pl.ds(base, TPW)], xbuf)
        pltpu.sync_copy(idx_hbm.at[wid], idx_v)
        c0 = pltpu.async_copy(xbuf, xs_hbm.at[idx_v.at[0]], sem0)
        c1 = pltpu.async_copy(xbuf, xs_hbm.at[idx_v.at[1]], sem1)
        c0.wait()
        c1.wait()

    return dispatch


def _sc_dispatch(hidden_states, idx3):
    return _sc_dispatch_kernel()(hidden_states, idx3)


@functools.lru_cache(maxsize=None)
def _sc_combine_kernel():
    mesh = plsc.VectorSubcoreMesh(core_axis_name="c", subcore_axis_name="s")

    @functools.partial(
        pl.kernel,
        mesh=mesh,
        out_type=[
            jax.ShapeDtypeStruct((T, H), jnp.float32),
            jax.ShapeDtypeStruct((T, H), jnp.float32),
        ],
        scratch_types=[
            pltpu.VMEM((K, TPW), jnp.int32),
            pltpu.VMEM((TPW, H), jnp.float32),
            pltpu.SemaphoreType.DMA,
        ],
    )
    def combine(y_hbm, idx_hbm, c0_hbm, c1_hbm, idx_v, ybuf, sem):
        wid = lax.axis_index("s") * 2 + lax.axis_index("c")
        base = wid * TPW
        pltpu.sync_copy(idx_hbm.at[wid], idx_v)
        pltpu.async_copy(y_hbm.at[idx_v.at[0]], ybuf, sem).wait()
        pltpu.sync_copy(ybuf, c0_hbm.at[pl.ds(base, TPW)])
        pltpu.async_copy(y_hbm.at[idx_v.at[1]], ybuf, sem).wait()
        pltpu.sync_copy(ybuf, c1_hbm.at[pl.ds(base, TPW)])

    return combine


def _sc_combine(ys, idx3):
    return _sc_combine_kernel()(ys, idx3)


def _group_body(te_ref, xs_ref, wg_ref, wu_ref, wd_ref, ys_ref):
    @pl.when(pl.program_id(0) < te_ref[NT])
    def _():
        x = xs_ref[...]
        g = jnp.dot(x, wg_ref[0], preferred_element_type=jnp.float32,
                    precision=jax.lax.Precision.DEFAULT)
        u = jnp.dot(x, wu_ref[0], preferred_element_type=jnp.float32,
                    precision=jax.lax.Precision.DEFAULT)
        ys_ref[...] = jnp.dot(g * jax.lax.logistic(g) * u, wd_ref[0],
                              preferred_element_type=jnp.float32,
                              precision=jax.lax.Precision.DEFAULT)


def _grouped_mlp(te_flat, xs, w_gate, w_up, w_down):
    grid_spec = pltpu.PrefetchScalarGridSpec(
        num_scalar_prefetch=1,
        grid=(NT,),
        in_specs=[
            pl.BlockSpec((B, H), lambda i, te: (i, 0)),
            pl.BlockSpec((1, H, F), lambda i, te: (te[i], 0, 0)),
            pl.BlockSpec((1, H, F), lambda i, te: (te[i], 0, 0)),
            pl.BlockSpec((1, F, H), lambda i, te: (te[i], 0, 0)),
        ],
        out_specs=pl.BlockSpec((B, H), lambda i, te: (i, 0)),
    )
    return pl.pallas_call(
        _group_body,
        grid_spec=grid_spec,
        out_shape=jax.ShapeDtypeStruct((P, H), jnp.float32),
        compiler_params=pltpu.CompilerParams(
            dimension_semantics=("arbitrary",)),
    )(te_flat, xs, w_gate, w_up, w_down)


NFB = SF // SFB
NF1 = 6


def _shared_part_body(nf, f0, is_last, has_prev):
    def body(h_ref, wg_ref, wu_ref, wd_ref, sgw_ref, *rest):
        if has_prev:
            prev_ref, out_ref = rest
        else:
            (out_ref,) = rest
        f = pl.program_id(1)
        x = h_ref[...]
        g = jnp.dot(x, wg_ref[...], preferred_element_type=jnp.float32,
                    precision=jax.lax.Precision.DEFAULT)
        u = jnp.dot(x, wu_ref[...], preferred_element_type=jnp.float32,
                    precision=jax.lax.Precision.DEFAULT)
        part = jnp.dot(g * jax.lax.logistic(g) * u, wd_ref[...],
                       preferred_element_type=jnp.float32,
                       precision=jax.lax.Precision.DEFAULT)
        init = prev_ref[...] if has_prev else 0.0
        acc = jnp.where(f == 0, init, out_ref[...]) + part

        if is_last:
            @pl.when(f == nf - 1)
            def _():
                gate = jnp.dot(x, sgw_ref[...], preferred_element_type=jnp.float32)
                out_ref[...] = acc * jax.lax.logistic(gate)

            @pl.when(f < nf - 1)
            def _():
                out_ref[...] = acc
        else:
            out_ref[...] = acc

    return body


def _shared_part(nf, f0, is_last, hidden_states, wg, wu, wd, sgw, prev=None):
    in_specs = [
        pl.BlockSpec((ST, H), lambda t, f: (t, 0)),
        pl.BlockSpec((H, SFB), lambda t, f: (0, f + f0)),
        pl.BlockSpec((H, SFB), lambda t, f: (0, f + f0)),
        pl.BlockSpec((SFB, H), lambda t, f: (f + f0, 0)),
        pl.BlockSpec((H, 1), lambda t, f: (0, 0)),
    ]
    args = [hidden_states, wg, wu, wd, sgw]
    if prev is not None:
        in_specs.append(pl.BlockSpec((ST, H), lambda t, f: (t, 0)))
        args.append(prev)
    return pl.pallas_call(
        _shared_part_body(nf, f0, is_last, prev is not None),
        grid=(T // ST, nf),
        in_specs=in_specs,
        out_specs=pl.BlockSpec((ST, H), lambda t, f: (t, 0)),
        out_shape=jax.ShapeDtypeStruct((T, H), jnp.float32),
        compiler_params=pltpu.CompilerParams(
            dimension_semantics=("parallel", "arbitrary")),
    )(*args)


def _final_body(sh_ref, c0_ref, c1_ref, tw_ref, out_ref):
    tw = tw_ref[...]
    out_ref[...] = (sh_ref[...] + tw[:, 0:1] * c0_ref[...]
                    + tw[:, 1:2] * c1_ref[...])


def _final(shared_out, c0, c1, tw):
    return pl.pallas_call(
        _final_body,
        grid=(T // RT,),
        in_specs=[
            pl.BlockSpec((RT, H), lambda t: (t, 0)),
            pl.BlockSpec((RT, H), lambda t: (t, 0)),
            pl.BlockSpec((RT, H), lambda t: (t, 0)),
            pl.BlockSpec((RT, K), lambda t: (t, 0)),
        ],
        out_specs=pl.BlockSpec((RT, H), lambda t: (t, 0)),
        out_shape=jax.ShapeDtypeStruct((T, H), jnp.float32),
    )(shared_out, c0, c1, tw)


def kernel(hidden_states, gate_w, w_gate, w_up, w_down,
           shared_w_gate, shared_w_up, shared_w_down, shared_gate_w):
    topk_ids, topk_w, slots, te = _route_meta(hidden_states, gate_w)

    idx3 = slots.reshape(NW, TPW, K).transpose(0, 2, 1)
    te_flat = te.reshape(NTE)

    xs = _sc_dispatch(hidden_states, idx3)
    ys = _grouped_mlp(te_flat, xs, w_gate, w_up, w_down)
    shared_out = _shared_part(NFB, 0, True, hidden_states, shared_w_gate,
                              shared_w_up, shared_w_down, shared_gate_w)
    c0, c1 = _sc_combine(ys, idx3)
    out = _final(shared_out, c0, c1, topk_w)
    return out, topk_ids
